# baseline (speedup 1.0000x reference)
import jax
import jax.numpy as jnp
from jax import lax
from jax.experimental import pallas as pl
from jax.experimental.pallas import tpu as pltpu
from jax.experimental.pallas import tpu_sc as plsc

NUM_TOKENS = 32768
NUM_EXPERTS = 64
TOP_K = 8
LANES = 16
NUM_CORES = 2
NUM_SUBCORES = 16
NUM_WORKERS = NUM_CORES * NUM_SUBCORES
TOK_PER_W = NUM_TOKENS // NUM_WORKERS
UNROLL = 8


def _merge_top8(a, ai, b, bi, lo_mask):
    rb = jnp.flip(b)
    rbi = jnp.flip(bi)
    d = jnp.where(lo_mask, a, rb)
    di = jnp.where(lo_mask, ai, rbi)
    return plsc.sort_key_val(d, di, descending=True)


def _tec_body(logits_hbm, w_hbm, id_hbm, id2_hbm, logits_v, w_v, id_v):
    wid = lax.axis_index("s") * NUM_CORES + lax.axis_index("c")
    base = wid * TOK_PER_W
    pltpu.sync_copy(logits_hbm.at[pl.ds(base * NUM_EXPERTS,
                                        TOK_PER_W * NUM_EXPERTS)], logits_v)

    iota = lax.iota(jnp.int32, LANES)
    lo_mask = iota < TOP_K

    def token_top8(t):
        sorted_chunks = []
        for q in range(NUM_EXPERTS // LANES):
            v = plsc.load_gather(
                logits_v, [t * NUM_EXPERTS + q * LANES + iota])
            sorted_chunks.append(
                plsc.sort_key_val(v, iota + q * LANES, descending=True))
        (a, ai), (b, bi), (c, ci), (d, di) = sorted_chunks
        m0, m0i = _merge_top8(a, ai, b, bi, lo_mask)
        m1, m1i = _merge_top8(c, ci, d, di, lo_mask)
        top, topi = _merge_top8(m0, m0i, m1, m1i, lo_mask)
        plsc.store_scatter(w_v, [t * TOP_K + iota], top, mask=lo_mask)
        plsc.store_scatter(id_v, [t * TOP_K + iota], topi, mask=lo_mask)

    @plsc.parallel_loop(0, TOK_PER_W, unroll=UNROLL)
    def pass1(t):
        token_top8(t)

    @plsc.parallel_loop(0, TOK_PER_W // LANES, unroll=2)
    def pass2(g):
        idx0 = g * (LANES * TOP_K) + iota * TOP_K
        vals = [plsc.load_gather(w_v, [idx0 + r]) for r in range(TOP_K)]
        es = [jnp.exp(v - vals[0]) for v in vals]
        s = es[0]
        for e in es[1:]:
            s = s + e
        for r in range(TOP_K):
            plsc.store_scatter(w_v, [idx0 + r], es[r] / s)

    pltpu.sync_copy(w_v, w_hbm.at[pl.ds(base * TOP_K, TOK_PER_W * TOP_K)])
    pltpu.sync_copy(id_v, id_hbm.at[pl.ds(base * TOP_K, TOK_PER_W * TOP_K)])
    pltpu.sync_copy(id_v, id2_hbm.at[pl.ds(base * TOP_K, TOK_PER_W * TOP_K)])


@jax.jit
def _select_topk(router_logits_fp32):
    mesh = plsc.VectorSubcoreMesh(
        core_axis_name="c", subcore_axis_name="s",
        num_cores=NUM_CORES, num_subcores=NUM_SUBCORES)
    fn = pl.kernel(
        _tec_body,
        out_type=(
            jax.ShapeDtypeStruct((NUM_TOKENS * TOP_K,), jnp.float32),
            jax.ShapeDtypeStruct((NUM_TOKENS * TOP_K,), jnp.int32),
            jax.ShapeDtypeStruct((NUM_TOKENS * TOP_K,), jnp.int32),
        ),
        mesh=mesh,
        compiler_params=pltpu.CompilerParams(needs_layout_passes=False),
        scratch_types=[
            pltpu.VMEM((TOK_PER_W * NUM_EXPERTS,), jnp.float32),
            pltpu.VMEM((TOK_PER_W * TOP_K,), jnp.float32),
            pltpu.VMEM((TOK_PER_W * TOP_K,), jnp.int32),
        ],
    )
    w_flat, id_flat, id2_flat = fn(router_logits_fp32.reshape(-1))
    return (w_flat.reshape(NUM_TOKENS, TOP_K),
            id_flat.reshape(NUM_TOKENS, TOP_K),
            id2_flat.reshape(NUM_TOKENS, TOP_K))


def kernel(router_logits_fp32, topk_ids, topk_weights):
    del topk_ids, topk_weights
    weights, ids, ids2 = _select_topk(router_logits_fp32)
    return (weights, ids, ids2)


# speedup vs baseline: 1.0414x; 1.0414x over previous
import jax
import jax.numpy as jnp
from jax import lax
from jax.experimental import pallas as pl
from jax.experimental.pallas import tpu as pltpu
from jax.experimental.pallas import tpu_sc as plsc

NUM_TOKENS = 32768
NUM_EXPERTS = 64
TOP_K = 8
LANES = 16
NUM_CORES = 2
NUM_SUBCORES = 16
NUM_WORKERS = NUM_CORES * NUM_SUBCORES
TOK_PER_W = NUM_TOKENS // NUM_WORKERS
UNROLL = 4


def _merge_top8(a, ai, b, bi, lo_mask):
    rb = jnp.flip(b)
    rbi = jnp.flip(bi)
    d = jnp.where(lo_mask, a, rb)
    di = jnp.where(lo_mask, ai, rbi)
    return plsc.sort_key_val(d, di, descending=True)


def _tec_body(logits_hbm, w_hbm, id_hbm, id2_hbm, logits_v, w_v, id_v):
    wid = lax.axis_index("s") * NUM_CORES + lax.axis_index("c")
    base = wid * TOK_PER_W
    pltpu.sync_copy(logits_hbm.at[pl.ds(base * NUM_EXPERTS,
                                        TOK_PER_W * NUM_EXPERTS)], logits_v)

    iota = lax.iota(jnp.int32, LANES)
    lo_mask = iota < TOP_K

    def token_top8(t):
        sorted_chunks = []
        for q in range(NUM_EXPERTS // LANES):
            v = plsc.load_gather(
                logits_v, [t * NUM_EXPERTS + q * LANES + iota])
            sorted_chunks.append(
                plsc.sort_key_val(v, iota + q * LANES, descending=True))
        (a, ai), (b, bi), (c, ci), (d, di) = sorted_chunks
        m0, m0i = _merge_top8(a, ai, b, bi, lo_mask)
        m1, m1i = _merge_top8(c, ci, d, di, lo_mask)
        top, topi = _merge_top8(m0, m0i, m1, m1i, lo_mask)
        plsc.store_scatter(w_v, [t * TOP_K + iota], top, mask=lo_mask)
        plsc.store_scatter(id_v, [t * TOP_K + iota], topi, mask=lo_mask)

    @plsc.parallel_loop(0, TOK_PER_W, unroll=UNROLL)
    def pass1(t):
        token_top8(t)

    @plsc.parallel_loop(0, TOK_PER_W // LANES, unroll=2)
    def pass2(g):
        idx0 = g * (LANES * TOP_K) + iota * TOP_K
        vals = [plsc.load_gather(w_v, [idx0 + r]) for r in range(TOP_K)]
        es = [jnp.exp(v - vals[0]) for v in vals]
        s = es[0]
        for e in es[1:]:
            s = s + e
        for r in range(TOP_K):
            plsc.store_scatter(w_v, [idx0 + r], es[r] / s)

    pltpu.sync_copy(w_v, w_hbm.at[pl.ds(base * TOP_K, TOK_PER_W * TOP_K)])
    pltpu.sync_copy(id_v, id_hbm.at[pl.ds(base * TOP_K, TOK_PER_W * TOP_K)])
    pltpu.sync_copy(id_v, id2_hbm.at[pl.ds(base * TOP_K, TOK_PER_W * TOP_K)])


@jax.jit
def _select_topk(router_logits_fp32):
    mesh = plsc.VectorSubcoreMesh(
        core_axis_name="c", subcore_axis_name="s",
        num_cores=NUM_CORES, num_subcores=NUM_SUBCORES)
    fn = pl.kernel(
        _tec_body,
        out_type=(
            jax.ShapeDtypeStruct((NUM_TOKENS * TOP_K,), jnp.float32),
            jax.ShapeDtypeStruct((NUM_TOKENS * TOP_K,), jnp.int32),
            jax.ShapeDtypeStruct((NUM_TOKENS * TOP_K,), jnp.int32),
        ),
        mesh=mesh,
        compiler_params=pltpu.CompilerParams(needs_layout_passes=False),
        scratch_types=[
            pltpu.VMEM((TOK_PER_W * NUM_EXPERTS,), jnp.float32),
            pltpu.VMEM((TOK_PER_W * TOP_K,), jnp.float32),
            pltpu.VMEM((TOK_PER_W * TOP_K,), jnp.int32),
        ],
    )
    w_flat, id_flat, id2_flat = fn(router_logits_fp32.reshape(-1))
    return (w_flat.reshape(NUM_TOKENS, TOP_K),
            id_flat.reshape(NUM_TOKENS, TOP_K),
            id2_flat.reshape(NUM_TOKENS, TOP_K))


def kernel(router_logits_fp32, topk_ids, topk_weights):
    del topk_ids, topk_weights
    weights, ids, ids2 = _select_topk(router_logits_fp32)
    return (weights, ids, ids2)


# speedup vs baseline: 1.2575x; 1.2075x over previous
import jax
import jax.numpy as jnp
from jax import lax
from jax.experimental import pallas as pl
from jax.experimental.pallas import tpu as pltpu
from jax.experimental.pallas import tpu_sc as plsc

NUM_TOKENS = 32768
NUM_EXPERTS = 64
TOP_K = 8
LANES = 16
NUM_CORES = 2
NUM_SUBCORES = 16
NUM_WORKERS = NUM_CORES * NUM_SUBCORES
TOK_PER_W = NUM_TOKENS // NUM_WORKERS
UNROLL = 4


def _merge_top8(a, ai, b, bi, lo_mask):
    rb = jnp.flip(b)
    rbi = jnp.flip(bi)
    d = jnp.where(lo_mask, a, rb)
    di = jnp.where(lo_mask, ai, rbi)
    return plsc.sort_key_val(d, di, descending=True)


def _tec_body(logits_hbm, w_hbm, id_hbm, logits_v, w_v, id_v):
    wid = lax.axis_index("s") * NUM_CORES + lax.axis_index("c")
    base = wid * TOK_PER_W
    pltpu.sync_copy(logits_hbm.at[pl.ds(base * NUM_EXPERTS,
                                        TOK_PER_W * NUM_EXPERTS)], logits_v)

    iota = lax.iota(jnp.int32, LANES)
    lo_mask = iota < TOP_K

    def token_top8(t):
        sorted_chunks = []
        for q in range(NUM_EXPERTS // LANES):
            v = plsc.load_gather(
                logits_v, [t * NUM_EXPERTS + q * LANES + iota])
            sorted_chunks.append(
                plsc.sort_key_val(v, iota + q * LANES, descending=True))
        (a, ai), (b, bi), (c, ci), (d, di) = sorted_chunks
        m0, m0i = _merge_top8(a, ai, b, bi, lo_mask)
        m1, m1i = _merge_top8(c, ci, d, di, lo_mask)
        top, topi = _merge_top8(m0, m0i, m1, m1i, lo_mask)
        plsc.store_scatter(w_v, [t * TOP_K + iota], top, mask=lo_mask)
        plsc.store_scatter(id_v, [t * TOP_K + iota], topi, mask=lo_mask)

    @plsc.parallel_loop(0, TOK_PER_W, unroll=UNROLL)
    def pass1(t):
        token_top8(t)

    @plsc.parallel_loop(0, TOK_PER_W // LANES, unroll=2)
    def pass2(g):
        idx0 = g * (LANES * TOP_K) + iota * TOP_K
        vals = [plsc.load_gather(w_v, [idx0 + r]) for r in range(TOP_K)]
        es = [jnp.exp(v - vals[0]) for v in vals]
        s = es[0]
        for e in es[1:]:
            s = s + e
        for r in range(TOP_K):
            plsc.store_scatter(w_v, [idx0 + r], es[r] / s)

    pltpu.sync_copy(w_v, w_hbm.at[pl.ds(base * TOP_K, TOK_PER_W * TOP_K)])
    pltpu.sync_copy(id_v, id_hbm.at[pl.ds(base * TOP_K, TOK_PER_W * TOP_K)])


@jax.jit
def _select_topk(router_logits_fp32):
    mesh = plsc.VectorSubcoreMesh(
        core_axis_name="c", subcore_axis_name="s",
        num_cores=NUM_CORES, num_subcores=NUM_SUBCORES)
    fn = pl.kernel(
        _tec_body,
        out_type=(
            jax.ShapeDtypeStruct((NUM_TOKENS * TOP_K,), jnp.float32),
            jax.ShapeDtypeStruct((NUM_TOKENS * TOP_K,), jnp.int32),
        ),
        mesh=mesh,
        compiler_params=pltpu.CompilerParams(needs_layout_passes=False),
        scratch_types=[
            pltpu.VMEM((TOK_PER_W * NUM_EXPERTS,), jnp.float32),
            pltpu.VMEM((TOK_PER_W * TOP_K,), jnp.float32),
            pltpu.VMEM((TOK_PER_W * TOP_K,), jnp.int32),
        ],
    )
    w_flat, id_flat = fn(router_logits_fp32.reshape(-1))
    return (w_flat.reshape(NUM_TOKENS, TOP_K),
            id_flat.reshape(NUM_TOKENS, TOP_K))


def kernel(router_logits_fp32, topk_ids, topk_weights):
    del topk_ids, topk_weights
    weights, ids = _select_topk(router_logits_fp32)
    return (weights, ids, ids)
